# baseline (speedup 1.0000x reference)
import functools

import jax
import jax.numpy as jnp
from jax import lax
from jax.experimental import pallas as pl
from jax.experimental.pallas import tpu as pltpu
from jax.experimental.pallas import tpu_sc as plsc

_NC = 2
_NS = 16
_NW = _NC * _NS
_L = 16
_TW = 128
_FLUSH = 128


def _make_sc_gather(B, N, H):
  n_tc = (N + _TW - 1) // _TW
  rng = 2 * ((n_tc + 2 * _NW - 1) // (2 * _NW))
  n_batch = rng // 2
  n_pair = n_batch // 2
  idx_chunk = 1024
  n_idx_chunks = B // idx_chunk
  nj = H // _L
  mesh = plsc.VectorSubcoreMesh(core_axis_name="c", subcore_axis_name="s")

  @functools.partial(
      pl.kernel,
      mesh=mesh,
      out_type=jax.ShapeDtypeStruct((B, 4 * H), jnp.float32),
      compiler_params=pltpu.CompilerParams(
          use_tc_tiling_on_sc=True,
          disable_bounds_checks=True,
          needs_layout_passes=False,
      ),
      scratch_types=[
          pltpu.VMEM((idx_chunk,), jnp.int32),
          pltpu.VMEM((B + _L,), jnp.int32),
          pltpu.VMEM((2, H, 2 * _TW), jnp.float32),
          pltpu.VMEM((2, H, 2 * _TW), jnp.float32),
          pltpu.VMEM((_FLUSH + _L, 4 * H), jnp.float32),
          pltpu.VMEM((_FLUSH + 2 * _L,), jnp.int32),
          pltpu.SemaphoreType.DMA,
          pltpu.SemaphoreType.DMA,
          pltpu.SemaphoreType.DMA,
      ],
  )
  def sc_gather(idx_hbm, mu_hbm, sp_hbm, fused_out,
                idx_v, wl, blk_mu, blk_sp, rowbuf, ilist,
                semf0, semf1, semo):
    wid = lax.axis_index("s") * _NC + lax.axis_index("c")
    lo = wid * rng
    hi = jnp.minimum(lo + rng, n_tc)
    iota16 = lax.broadcasted_iota(jnp.int32, (_L,), 0)
    iotas = [iota16 + j * _L for j in range(nj)]
    zeros16 = jnp.zeros((_L,), jnp.int32)
    fsems = (semf0, semf1)

    def scan_chunk(c, cnt):
      pltpu.sync_copy(idx_hbm.at[pl.ds(c * idx_chunk, idx_chunk)], idx_v)

      def scan_vec(t, cnt):
        v = idx_v[pl.ds(t * _L, _L)]
        gi = c * idx_chunk + t * _L + iota16
        tc_v = lax.shift_right_logical(v, 7)
        m = (tc_v >= lo) & (tc_v < hi)
        ent = gi | ((v & (_TW - 1)) << 14) | ((tc_v - lo) << 21)
        pref = plsc.cumsum(m.astype(jnp.int32))
        plsc.store_compressed(wl.at[pl.ds(cnt, _L)], ent, mask=m)
        return cnt + pref[_L - 1]

      return lax.fori_loop(0, idx_chunk // _L, scan_vec, cnt)

    cnt = lax.fori_loop(0, n_idx_chunks, scan_chunk, jnp.int32(0))
    n_scan = lax.shift_right_logical(cnt + _L - 1, 4)

    def fire(ph, b):
      tcb = jnp.minimum(lo + 2 * b, n_tc - 2)
      off = pl.multiple_of(tcb * _TW, _TW)
      pltpu.async_copy(mu_hbm.at[:, pl.ds(off, 2 * _TW)], blk_mu.at[ph],
                       fsems[ph])
      pltpu.async_copy(sp_hbm.at[:, pl.ds(off, 2 * _TW)], blk_sp.at[ph],
                       fsems[ph])

    def drain(ph):
      pltpu.make_async_copy(
          mu_hbm.at[:, pl.ds(0, 2 * _TW)], blk_mu.at[ph], fsems[ph]).wait()
      pltpu.make_async_copy(
          mu_hbm.at[:, pl.ds(0, 2 * _TW)], blk_sp.at[ph], fsems[ph]).wait()

    def flush(rowcnt):
      @pl.when(rowcnt >= _FLUSH)
      def _():
        for c in range(_FLUSH // _L):
          iv = ilist[pl.ds(c * _L, _L)]
          pltpu.async_copy(rowbuf.at[pl.ds(c * _L, _L)],
                           fused_out.at[iv], semo)
        pltpu.make_async_copy(fused_out.at[pl.ds(0, _FLUSH)],
                              rowbuf.at[pl.ds(0, _FLUSH)], semo).wait()
        for r in range(_L):
          for j in range(4 * H // _L):
            rowbuf[r, pl.ds(j * _L, _L)] = rowbuf[_FLUSH + r,
                                                  pl.ds(j * _L, _L)]
        m1 = ilist[pl.ds(_FLUSH, _L)]
        m2 = ilist[pl.ds(_FLUSH + _L, _L)]
        ilist[pl.ds(0, _L)] = m1
        ilist[pl.ds(_L, _L)] = m2

      return jnp.where(rowcnt >= _FLUSH, rowcnt - _FLUSH, rowcnt)

    def extract(b, ph, carry):
      tcb = jnp.minimum(lo + 2 * b, n_tc - 2)

      def ext_chunk(t, carry):
        rowcnt, lasti = carry
        e = wl[pl.ds(t * _L, _L)]
        posi = t * _L + iota16
        i_v = e & 0x3FFF
        rr_v = lax.shift_right_logical(e, 14) & (_TW - 1)
        tcl_v = lax.shift_right_logical(e, 21) & 0xFF
        mb = (posi < cnt) & (tcl_v >= 2 * b) & (tcl_v < 2 * b + 2)
        mi = mb.astype(jnp.int32)
        pref = plsc.cumsum(mi)
        nm = pref[_L - 1]

        @pl.when(nm > 0)
        def _():
          for k in range(_L):
            @pl.when(mi[k] == 1)
            def _():
              col = (tcl_v[k] + lo - tcb) * _TW + rr_v[k]
              ri = zeros16 + col
              pos = rowcnt + pref[k] - 1
              for j in range(nj):
                mvec = plsc.load_gather(blk_mu.at[ph], [iotas[j], ri])
                pvec = plsc.load_gather(blk_sp.at[ph], [iotas[j], ri])
                rowbuf[pos, pl.ds(j * _L, _L)] = mvec
                rowbuf[pos, pl.ds(H + j * _L, _L)] = pvec
                rowbuf[pos, pl.ds(2 * H + j * _L, _L)] = pvec * pvec

          plsc.store_compressed(ilist.at[pl.ds(rowcnt, _L)], i_v, mask=mb)

        lsel = jnp.where(mb & (pref == nm), i_v, 0)
        lasti = jnp.where(nm > 0, jnp.max(lsel), lasti)
        rowcnt = flush(rowcnt + nm)
        return (rowcnt, lasti)

      return lax.fori_loop(0, n_scan, ext_chunk, carry)

    def pair(b, carry):
      fire(0, b)
      drain(0)
      return extract(b, 0, carry)

    rowcnt, lasti = lax.fori_loop(0, n_batch, pair, (jnp.int32(0),
                                                     jnp.int32(0)))

    @pl.when(rowcnt > 0)
    def _():
      lastpos = rowcnt - 1
      for c in range(_FLUSH // _L):
        curv = ilist[pl.ds(c * _L, _L)]
        posi = c * _L + iota16
        ilist[pl.ds(c * _L, _L)] = jnp.where(posi < rowcnt, curv,
                                             zeros16 + lasti)
      lrow = [rowbuf[lastpos, pl.ds(j * _L, _L)] for j in range(3 * nj)]

      def padrow(t, carry):
        @pl.when(t >= rowcnt)
        def _():
          for j in range(3 * nj):
            rowbuf[t, pl.ds(j * _L, _L)] = lrow[j]
        return carry

      lax.fori_loop(0, _FLUSH, padrow, 0)
      for c in range(_FLUSH // _L):
        iv = ilist[pl.ds(c * _L, _L)]
        pltpu.async_copy(rowbuf.at[pl.ds(c * _L, _L)],
                         fused_out.at[iv], semo)
      pltpu.make_async_copy(fused_out.at[pl.ds(0, _FLUSH)],
                            rowbuf.at[pl.ds(0, _FLUSH)], semo).wait()

  return sc_gather


def kernel(X, indexes, mu, Sigma_param):
  del X
  B = indexes.shape[0]
  N, H = mu.shape
  idx = indexes.astype(jnp.int32)
  fused = _make_sc_gather(B, N, H)(idx, mu.T, Sigma_param.T)
  return (fused[:, :H], fused[:, H:2 * H], fused[:, 2 * H:3 * H])


# speedup vs baseline: 1.3216x; 1.3216x over previous
import functools

import jax
import jax.numpy as jnp
from jax import lax
from jax.experimental import pallas as pl
from jax.experimental.pallas import tpu as pltpu
from jax.experimental.pallas import tpu_sc as plsc

_NC = 2
_NS = 16
_NW = _NC * _NS
_L = 16
_TW = 128
_FLUSH = 128


def _make_sc_gather(B, N, H):
  n_tc = (N + _TW - 1) // _TW
  rng = 2 * ((n_tc + 2 * _NW - 1) // (2 * _NW))
  n_batch = rng // 2
  n_pair = n_batch // 2
  idx_chunk = 1024
  n_idx_chunks = B // idx_chunk
  nj = H // _L
  mesh = plsc.VectorSubcoreMesh(core_axis_name="c", subcore_axis_name="s")

  @functools.partial(
      pl.kernel,
      mesh=mesh,
      out_type=jax.ShapeDtypeStruct((B, 4 * H), jnp.float32),
      compiler_params=pltpu.CompilerParams(
          use_tc_tiling_on_sc=True,
          disable_bounds_checks=True,
          needs_layout_passes=False,
      ),
      scratch_types=[
          pltpu.VMEM((idx_chunk,), jnp.int32),
          pltpu.VMEM((B + _L,), jnp.int32),
          pltpu.VMEM((2, H, 2 * _TW), jnp.float32),
          pltpu.VMEM((2, H, 2 * _TW), jnp.float32),
          pltpu.VMEM((_FLUSH + _L, 4 * H), jnp.float32),
          pltpu.VMEM((_FLUSH + 2 * _L,), jnp.int32),
          pltpu.SemaphoreType.DMA,
          pltpu.SemaphoreType.DMA,
          pltpu.SemaphoreType.DMA,
      ],
  )
  def sc_gather(idx_hbm, mu_hbm, sp_hbm, fused_out,
                idx_v, wl, blk_mu, blk_sp, rowbuf, ilist,
                semf0, semf1, semo):
    wid = lax.axis_index("s") * _NC + lax.axis_index("c")
    lo = wid * rng
    hi = jnp.minimum(lo + rng, n_tc)
    iota16 = lax.broadcasted_iota(jnp.int32, (_L,), 0)
    iotas = [iota16 + j * _L for j in range(nj)]
    zeros16 = jnp.zeros((_L,), jnp.int32)
    fsems = (semf0, semf1)

    def scan_chunk(c, cnt):
      pltpu.sync_copy(idx_hbm.at[pl.ds(c * idx_chunk, idx_chunk)], idx_v)

      def scan_vec(t, cnt):
        v = idx_v[pl.ds(t * _L, _L)]
        gi = c * idx_chunk + t * _L + iota16
        tc_v = lax.shift_right_logical(v, 7)
        m = (tc_v >= lo) & (tc_v < hi)
        ent = gi | ((v & (_TW - 1)) << 14) | ((tc_v - lo) << 21)
        pref = plsc.cumsum(m.astype(jnp.int32))
        plsc.store_compressed(wl.at[pl.ds(cnt, _L)], ent, mask=m)
        return cnt + pref[_L - 1]

      return lax.fori_loop(0, idx_chunk // _L, scan_vec, cnt)

    cnt = lax.fori_loop(0, n_idx_chunks, scan_chunk, jnp.int32(0))
    n_scan = lax.shift_right_logical(cnt + _L - 1, 4)

    def fire(ph, b):
      tcb = jnp.minimum(lo + 2 * b, n_tc - 2)
      off = pl.multiple_of(tcb * _TW, _TW)
      pltpu.async_copy(mu_hbm.at[:, pl.ds(off, 2 * _TW)], blk_mu.at[ph],
                       fsems[ph])
      pltpu.async_copy(sp_hbm.at[:, pl.ds(off, 2 * _TW)], blk_sp.at[ph],
                       fsems[ph])

    def drain(ph):
      pltpu.make_async_copy(
          mu_hbm.at[:, pl.ds(0, 2 * _TW)], blk_mu.at[ph], fsems[ph]).wait()
      pltpu.make_async_copy(
          mu_hbm.at[:, pl.ds(0, 2 * _TW)], blk_sp.at[ph], fsems[ph]).wait()

    def flush(rowcnt):
      @pl.when(rowcnt >= _FLUSH)
      def _():
        for c in range(_FLUSH // _L):
          iv = ilist[pl.ds(c * _L, _L)]
          pltpu.async_copy(rowbuf.at[pl.ds(c * _L, _L)],
                           fused_out.at[iv], semo)
        pltpu.make_async_copy(fused_out.at[pl.ds(0, _FLUSH)],
                              rowbuf.at[pl.ds(0, _FLUSH)], semo).wait()
        for r in range(_L):
          for j in range(4 * H // _L):
            rowbuf[r, pl.ds(j * _L, _L)] = rowbuf[_FLUSH + r,
                                                  pl.ds(j * _L, _L)]
        m1 = ilist[pl.ds(_FLUSH, _L)]
        m2 = ilist[pl.ds(_FLUSH + _L, _L)]
        ilist[pl.ds(0, _L)] = m1
        ilist[pl.ds(_L, _L)] = m2

      return jnp.where(rowcnt >= _FLUSH, rowcnt - _FLUSH, rowcnt)

    def extract(b, ph, carry):
      tcb = jnp.minimum(lo + 2 * b, n_tc - 2)

      def ext_chunk(t, carry):
        rowcnt, lasti = carry
        e = wl[pl.ds(t * _L, _L)]
        posi = t * _L + iota16
        i_v = e & 0x3FFF
        rr_v = lax.shift_right_logical(e, 14) & (_TW - 1)
        tcl_v = lax.shift_right_logical(e, 21) & 0xFF
        mb = (posi < cnt) & (tcl_v >= 2 * b) & (tcl_v < 2 * b + 2)
        mi = mb.astype(jnp.int32)
        pref = plsc.cumsum(mi)
        nm = pref[_L - 1]

        @pl.when(nm > 0)
        def _():
          for k in range(_L):
            @pl.when(mi[k] == 1)
            def _():
              col = (tcl_v[k] + lo - tcb) * _TW + rr_v[k]
              ri = zeros16 + col
              pos = rowcnt + pref[k] - 1
              for j in range(nj):
                mvec = plsc.load_gather(blk_mu.at[ph], [iotas[j], ri])
                pvec = plsc.load_gather(blk_sp.at[ph], [iotas[j], ri])
                rowbuf[pos, pl.ds(j * _L, _L)] = mvec
                rowbuf[pos, pl.ds(H + j * _L, _L)] = pvec
                rowbuf[pos, pl.ds(2 * H + j * _L, _L)] = pvec * pvec

          plsc.store_compressed(ilist.at[pl.ds(rowcnt, _L)], i_v, mask=mb)

        lsel = jnp.where(mb & (pref == nm), i_v, 0)
        lasti = jnp.where(nm > 0, jnp.max(lsel), lasti)
        rowcnt = flush(rowcnt + nm)
        return (rowcnt, lasti)

      return lax.fori_loop(0, n_scan, ext_chunk, carry)

    fire(0, 0)

    def pair(pp, carry):
      b0 = 2 * pp
      fire(1, jnp.minimum(b0 + 1, n_batch - 1))
      drain(0)
      carry = extract(b0, 0, carry)
      fire(0, jnp.minimum(b0 + 2, n_batch - 1))
      drain(1)
      carry = extract(b0 + 1, 1, carry)
      return carry

    rowcnt, lasti = lax.fori_loop(0, n_pair + 1, pair, (jnp.int32(0),
                                                        jnp.int32(0)))
    drain(0)

    @pl.when(rowcnt > 0)
    def _():
      lastpos = rowcnt - 1
      for c in range(_FLUSH // _L):
        curv = ilist[pl.ds(c * _L, _L)]
        posi = c * _L + iota16
        ilist[pl.ds(c * _L, _L)] = jnp.where(posi < rowcnt, curv,
                                             zeros16 + lasti)
      lrow = [rowbuf[lastpos, pl.ds(j * _L, _L)] for j in range(3 * nj)]

      def padrow(t, carry):
        @pl.when(t >= rowcnt)
        def _():
          for j in range(3 * nj):
            rowbuf[t, pl.ds(j * _L, _L)] = lrow[j]
        return carry

      lax.fori_loop(0, _FLUSH, padrow, 0)
      for c in range(_FLUSH // _L):
        iv = ilist[pl.ds(c * _L, _L)]
        pltpu.async_copy(rowbuf.at[pl.ds(c * _L, _L)],
                         fused_out.at[iv], semo)
      pltpu.make_async_copy(fused_out.at[pl.ds(0, _FLUSH)],
                            rowbuf.at[pl.ds(0, _FLUSH)], semo).wait()

  return sc_gather


def kernel(X, indexes, mu, Sigma_param):
  del X
  B = indexes.shape[0]
  N, H = mu.shape
  idx = indexes.astype(jnp.int32)
  fused = _make_sc_gather(B, N, H)(idx, mu.T, Sigma_param.T)
  return (fused[:, :H], fused[:, H:2 * H], fused[:, 2 * H:3 * H])


# speedup vs baseline: 1.4890x; 1.1267x over previous
import functools

import jax
import jax.numpy as jnp
from jax import lax
from jax.experimental import pallas as pl
from jax.experimental.pallas import tpu as pltpu
from jax.experimental.pallas import tpu_sc as plsc

_NC = 2
_NS = 16
_NW = _NC * _NS
_L = 16
_TW = 128
_FLUSH = 128


def _popcount(mask):
  p = plsc.all_reduce_population_count(mask)
  return p if p.ndim == 0 else p[0]


def _make_sc_gather(B, N, H):
  n_tc = (N + _TW - 1) // _TW
  rng = 2 * ((n_tc + 2 * _NW - 1) // (2 * _NW))
  n_batch = rng // 2
  n_pair = n_batch // 2
  idx_chunk = 1024
  n_idx_chunks = B // idx_chunk
  nj = H // _L
  mesh = plsc.VectorSubcoreMesh(core_axis_name="c", subcore_axis_name="s")

  @functools.partial(
      pl.kernel,
      mesh=mesh,
      out_type=jax.ShapeDtypeStruct((B, 4 * H), jnp.float32),
      compiler_params=pltpu.CompilerParams(
          use_tc_tiling_on_sc=True,
          disable_bounds_checks=True,
          needs_layout_passes=False,
      ),
      scratch_types=[
          pltpu.VMEM((idx_chunk,), jnp.int32),
          pltpu.VMEM((B + _L,), jnp.int32),
          pltpu.VMEM((2, H, 2 * _TW), jnp.float32),
          pltpu.VMEM((2, H, 2 * _TW), jnp.float32),
          pltpu.VMEM((_FLUSH + _L, 4 * H), jnp.float32),
          pltpu.VMEM((_FLUSH + 2 * _L,), jnp.int32),
          pltpu.SemaphoreType.DMA,
          pltpu.SemaphoreType.DMA,
          pltpu.SemaphoreType.DMA,
      ],
  )
  def sc_gather(idx_hbm, mu_hbm, sp_hbm, fused_out,
                idx_v, wl, blk_mu, blk_sp, rowbuf, ilist,
                semf0, semf1, semo):
    wid = lax.axis_index("s") * _NC + lax.axis_index("c")
    lo = wid * rng
    hi = jnp.minimum(lo + rng, n_tc)
    iota16 = lax.broadcasted_iota(jnp.int32, (_L,), 0)
    iotas = [iota16 + j * _L for j in range(nj)]
    zeros16 = jnp.zeros((_L,), jnp.int32)
    fsems = (semf0, semf1)

    def scan_chunk(c, cnt):
      pltpu.sync_copy(idx_hbm.at[pl.ds(c * idx_chunk, idx_chunk)], idx_v)

      def scan_vec(t, cnt):
        v = idx_v[pl.ds(t * _L, _L)]
        gi = c * idx_chunk + t * _L + iota16
        tc_v = lax.shift_right_logical(v, 7)
        m = (tc_v >= lo) & (tc_v < hi)
        ent = gi | ((v & (_TW - 1)) << 14) | ((tc_v - lo) << 21)
        nm = _popcount(m)

        @pl.when(nm > 0)
        def _():
          plsc.store_compressed(wl.at[pl.ds(cnt, _L)], ent, mask=m)

        return cnt + nm

      return lax.fori_loop(0, idx_chunk // _L, scan_vec, cnt)

    cnt = lax.fori_loop(0, n_idx_chunks, scan_chunk, jnp.int32(0))
    n_scan = lax.shift_right_logical(cnt + _L - 1, 4)

    def fire(ph, b):
      tcb = jnp.minimum(lo + 2 * b, n_tc - 2)
      off = pl.multiple_of(tcb * _TW, _TW)
      pltpu.async_copy(mu_hbm.at[:, pl.ds(off, 2 * _TW)], blk_mu.at[ph],
                       fsems[ph])
      pltpu.async_copy(sp_hbm.at[:, pl.ds(off, 2 * _TW)], blk_sp.at[ph],
                       fsems[ph])

    def drain(ph):
      pltpu.make_async_copy(
          mu_hbm.at[:, pl.ds(0, 2 * _TW)], blk_mu.at[ph], fsems[ph]).wait()
      pltpu.make_async_copy(
          mu_hbm.at[:, pl.ds(0, 2 * _TW)], blk_sp.at[ph], fsems[ph]).wait()

    def flush(rowcnt):
      @pl.when(rowcnt >= _FLUSH)
      def _():
        for c in range(_FLUSH // _L):
          iv = ilist[pl.ds(c * _L, _L)]
          pltpu.async_copy(rowbuf.at[pl.ds(c * _L, _L)],
                           fused_out.at[iv], semo)
        pltpu.make_async_copy(fused_out.at[pl.ds(0, _FLUSH)],
                              rowbuf.at[pl.ds(0, _FLUSH)], semo).wait()
        for r in range(_L):
          for j in range(4 * H // _L):
            rowbuf[r, pl.ds(j * _L, _L)] = rowbuf[_FLUSH + r,
                                                  pl.ds(j * _L, _L)]
        m1 = ilist[pl.ds(_FLUSH, _L)]
        m2 = ilist[pl.ds(_FLUSH + _L, _L)]
        ilist[pl.ds(0, _L)] = m1
        ilist[pl.ds(_L, _L)] = m2

      return jnp.where(rowcnt >= _FLUSH, rowcnt - _FLUSH, rowcnt)

    def extract(b, ph, rowcnt):
      tcb = jnp.minimum(lo + 2 * b, n_tc - 2)

      def ext_chunk(t, rowcnt):
        e = wl[pl.ds(t * _L, _L)]
        posi = t * _L + iota16
        tcl_v = lax.shift_right_logical(e, 21) & 0xFF
        mb = (posi < cnt) & (tcl_v >= 2 * b) & (tcl_v < 2 * b + 2)
        nm = _popcount(mb)

        @pl.when(nm > 0)
        def _():
          i_v = e & 0x3FFF
          rr_v = lax.shift_right_logical(e, 14) & (_TW - 1)
          mi = mb.astype(jnp.int32)
          pref = plsc.cumsum(mi)
          for k in range(_L):
            @pl.when(mi[k] == 1)
            def _():
              col = (tcl_v[k] + lo - tcb) * _TW + rr_v[k]
              ri = zeros16 + col
              pos = rowcnt + pref[k] - 1
              for j in range(nj):
                mvec = plsc.load_gather(blk_mu.at[ph], [iotas[j], ri])
                pvec = plsc.load_gather(blk_sp.at[ph], [iotas[j], ri])
                rowbuf[pos, pl.ds(j * _L, _L)] = mvec
                rowbuf[pos, pl.ds(H + j * _L, _L)] = pvec
                rowbuf[pos, pl.ds(2 * H + j * _L, _L)] = pvec * pvec

          plsc.store_compressed(ilist.at[pl.ds(rowcnt, _L)], i_v, mask=mb)

        return flush(rowcnt + nm)

      return lax.fori_loop(0, n_scan, ext_chunk, rowcnt)

    fire(0, 0)

    def pair(pp, rowcnt):
      b0 = 2 * pp
      fire(1, jnp.minimum(b0 + 1, n_batch - 1))
      drain(0)
      rowcnt = extract(b0, 0, rowcnt)
      fire(0, jnp.minimum(b0 + 2, n_batch - 1))
      drain(1)
      rowcnt = extract(b0 + 1, 1, rowcnt)
      return rowcnt

    rowcnt = lax.fori_loop(0, n_pair + 1, pair, jnp.int32(0))
    drain(0)

    @pl.when(rowcnt > 0)
    def _():
      head = ilist[pl.ds(0, _L)]
      for c in range(_FLUSH // _L):
        curv = ilist[pl.ds(c * _L, _L)]
        posi = c * _L + iota16
        ilist[pl.ds(c * _L, _L)] = jnp.where(posi < rowcnt, curv,
                                             zeros16 + head[0])
      lrow = [rowbuf[0, pl.ds(j * _L, _L)] for j in range(3 * nj)]

      def padrow(t, carry):
        @pl.when(t >= rowcnt)
        def _():
          for j in range(3 * nj):
            rowbuf[t, pl.ds(j * _L, _L)] = lrow[j]
        return carry

      lax.fori_loop(0, _FLUSH, padrow, 0)
      for c in range(_FLUSH // _L):
        iv = ilist[pl.ds(c * _L, _L)]
        pltpu.async_copy(rowbuf.at[pl.ds(c * _L, _L)],
                         fused_out.at[iv], semo)
      pltpu.make_async_copy(fused_out.at[pl.ds(0, _FLUSH)],
                            rowbuf.at[pl.ds(0, _FLUSH)], semo).wait()

  return sc_gather


def kernel(X, indexes, mu, Sigma_param):
  del X
  B = indexes.shape[0]
  N, H = mu.shape
  idx = indexes.astype(jnp.int32)
  fused = _make_sc_gather(B, N, H)(idx, mu.T, Sigma_param.T)
  return (fused[:, :H], fused[:, H:2 * H], fused[:, 2 * H:3 * H])


# speedup vs baseline: 1.6976x; 1.1401x over previous
import functools

import jax
import jax.numpy as jnp
from jax import lax
from jax.experimental import pallas as pl
from jax.experimental.pallas import tpu as pltpu
from jax.experimental.pallas import tpu_sc as plsc

_NC = 2
_NS = 16
_NW = _NC * _NS
_L = 16
_TW = 128


def _make_sc_gather(B, N, H):
  b_per_w = B // _NW
  n_tc = (N + _TW - 1) // _TW
  n_chunks = b_per_w // _L
  mesh = plsc.VectorSubcoreMesh(core_axis_name="c", subcore_axis_name="s")

  @functools.partial(
      pl.kernel,
      mesh=mesh,
      out_type=jax.ShapeDtypeStruct((B, 4 * H), jnp.float32),
      compiler_params=pltpu.CompilerParams(
          use_tc_tiling_on_sc=True,
          disable_bounds_checks=True,
          needs_layout_passes=False,
      ),
      scratch_types=[
          pltpu.VMEM((b_per_w + _L,), jnp.int32),
          pltpu.VMEM((2, 2, H, _TW), jnp.float32),
          pltpu.VMEM((2, 2, H, _TW), jnp.float32),
          pltpu.VMEM((2 * _L, 4 * H), jnp.float32),
          pltpu.SemaphoreType.DMA,
          pltpu.SemaphoreType.DMA,
          pltpu.SemaphoreType.DMA,
      ],
  )
  def sc_gather(idx_hbm, mu_hbm, sp_hbm, fused_out,
                idx_v, blk_mu, blk_sp, rowbuf, semf0, semf1, semo):
    wid = lax.axis_index("s") * _NC + lax.axis_index("c")
    base = wid * b_per_w
    iotas = [
        lax.broadcasted_iota(jnp.int32, (_L,), 0) + j * _L
        for j in range(H // _L)
    ]
    zeros16 = jnp.zeros((_L,), jnp.int32)
    fsems = (semf0, semf1)

    pltpu.sync_copy(idx_hbm.at[pl.ds(base, b_per_w)],
                    idx_v.at[pl.ds(0, b_per_w)])

    def fire(ph, pp, s):
      tc = jnp.minimum(lax.shift_right_logical(s, 7), n_tc - 1)
      off = pl.multiple_of(tc * _TW, _TW)
      pltpu.async_copy(mu_hbm.at[:, pl.ds(off, _TW)], blk_mu.at[ph, pp],
                       fsems[ph])
      pltpu.async_copy(sp_hbm.at[:, pl.ds(off, _TW)], blk_sp.at[ph, pp],
                       fsems[ph])

    def extract(ph, pp, s, rowpos):
      rr = jnp.bitwise_and(s, _TW - 1)
      ri = zeros16 + rr
      for j in range(H // _L):
        m = plsc.load_gather(blk_mu.at[ph, pp], [iotas[j], ri])
        p = plsc.load_gather(blk_sp.at[ph, pp], [iotas[j], ri])
        rowbuf[rowpos, pl.ds(j * _L, _L)] = m
        rowbuf[rowpos, pl.ds(H + j * _L, _L)] = p
        rowbuf[rowpos, pl.ds(2 * H + j * _L, _L)] = p * p

    v0 = idx_v[pl.ds(0, _L)]
    fire(0, 0, v0[0])
    fire(0, 1, v0[1])

    def body(c, carry):
      v = idx_v[pl.ds(c * _L, _L)]
      vn = idx_v[pl.ds(c * _L + _L, _L)]
      rbase = jnp.bitwise_and(c, 1) * _L
      for g in range(8):
        ph = g & 1
        nph = (g + 1) & 1
        if g < 7:
          fire(nph, 0, v[2 * g + 2])
          fire(nph, 1, v[2 * g + 3])
        else:
          @pl.when(c < n_chunks - 1)
          def _():
            fire(nph, 0, vn[0])
            fire(nph, 1, vn[1])
        pltpu.make_async_copy(
            mu_hbm.at[:, pl.ds(0, _TW)], blk_mu.at[ph, 0], fsems[ph]).wait()
        pltpu.make_async_copy(
            mu_hbm.at[:, pl.ds(0, _TW)], blk_mu.at[ph, 1], fsems[ph]).wait()
        pltpu.make_async_copy(
            mu_hbm.at[:, pl.ds(0, _TW)], blk_sp.at[ph, 0], fsems[ph]).wait()
        pltpu.make_async_copy(
            mu_hbm.at[:, pl.ds(0, _TW)], blk_sp.at[ph, 1], fsems[ph]).wait()
        extract(ph, 0, v[2 * g], rbase + 2 * g)
        extract(ph, 1, v[2 * g + 1], rbase + 2 * g + 1)

      @pl.when(jnp.bitwise_and(c, 1) == 1)
      def _():
        off = pl.multiple_of(base + (c - 1) * _L, 2 * _L)
        cp = pltpu.async_copy(rowbuf, fused_out.at[pl.ds(off, 2 * _L)], semo)
        cp.wait()

      return carry

    lax.fori_loop(0, n_chunks, body, 0)

  return sc_gather


def kernel(X, indexes, mu, Sigma_param):
  del X
  B = indexes.shape[0]
  N, H = mu.shape
  idx = indexes.astype(jnp.int32)
  fused = _make_sc_gather(B, N, H)(idx, mu.T, Sigma_param.T)
  return (fused[:, :H], fused[:, H:2 * H], fused[:, 2 * H:3 * H])
